# baseline (speedup 1.0000x reference)
import functools

import jax
import jax.numpy as jnp
from jax import lax
from jax.experimental import pallas as pl
from jax.experimental.pallas import tpu as pltpu
from jax.experimental.pallas import tpu_sc as plsc

_N = 500000
_C = 100
_CHUNK = 400
_CW = _CHUNK * _C
_NCH = _N // _CHUNK
_NW = 32
_BASE = _NCH // _NW
_EXTRA = _NCH - _BASE * _NW
_MAXC = _BASE + 1
_LBUF = _MAXC * _CHUNK

_mesh = plsc.VectorSubcoreMesh(core_axis_name="c", subcore_axis_name="s")


@functools.partial(
    pl.kernel,
    mesh=_mesh,
    out_type=jax.ShapeDtypeStruct((_N * _C,), jnp.float32),
    scratch_types=[
        pltpu.VMEM((_CW,), jnp.float32),
        pltpu.VMEM((_LBUF,), jnp.int32),
    ],
    compiler_params=pltpu.CompilerParams(needs_layout_passes=False),
)
def _sc_onehot(labels_hbm, out_hbm, buf, lab_v):
    cid = lax.axis_index("c")
    sid = lax.axis_index("s")
    wid = sid * 2 + cid

    start = wid * _BASE + jnp.minimum(wid, _EXTRA)
    nch = jnp.where(wid < _EXTRA, _BASE + 1, _BASE)
    row0 = start * _CHUNK
    wbase = jnp.minimum(row0, _N - _LBUF)
    loff = row0 - wbase
    pltpu.sync_copy(labels_hbm.at[pl.ds(wbase, _LBUF)], lab_v)

    def _zero(i, carry):
        buf[pl.ds(i * 16, 16)] = jnp.zeros((16,), jnp.float32)
        return carry

    lax.fori_loop(0, _CW // 16, _zero, 0)

    ones = jnp.ones((16,), jnp.float32)
    zeros = jnp.zeros((16,), jnp.float32)
    iota = lax.iota(jnp.int32, 16)

    def _chunk(k, carry):
        @pl.when(k < nch)
        def _():
            lbase = loff + k * _CHUNK
            idxs = []
            for j in range(_CHUNK // 16):
                lab16 = lab_v[pl.ds(lbase + j * 16, 16)]
                lab16 = jnp.where(lab16 < 0, 55, lab16)
                idx = (j * 16 + iota) * _C + lab16
                plsc.store_scatter(buf, [idx], ones)
                idxs.append(idx)
            pltpu.sync_copy(buf, out_hbm.at[pl.ds((start + k) * _CW, _CW)])
            for idx in idxs:
                plsc.store_scatter(buf, [idx], zeros)
        return carry

    lax.fori_loop(0, _MAXC, _chunk, 0)


def kernel(x, labels):
    out = _sc_onehot(labels)
    return out.reshape(_N, _C)


# speedup vs baseline: 1.0067x; 1.0067x over previous
import functools

import jax
import jax.numpy as jnp
from jax import lax
from jax.experimental import pallas as pl
from jax.experimental.pallas import tpu as pltpu
from jax.experimental.pallas import tpu_sc as plsc

_N = 500000
_C = 100
_CHUNK = 400
_CW = _CHUNK * _C
_NCH = _N // _CHUNK
_NW = 32
_BASE = _NCH // _NW
_EXTRA = _NCH - _BASE * _NW
_MAXC = _BASE + 1
_LBUF = _MAXC * _CHUNK

_mesh = plsc.VectorSubcoreMesh(core_axis_name="c", subcore_axis_name="s")


@functools.partial(
    pl.kernel,
    mesh=_mesh,
    out_type=jax.ShapeDtypeStruct((_N * _C,), jnp.float32),
    scratch_types=[
        pltpu.VMEM((_CW,), jnp.float32),
        pltpu.VMEM((_CW,), jnp.float32),
        pltpu.VMEM((_LBUF,), jnp.int32),
        pltpu.SemaphoreType.DMA,
        pltpu.SemaphoreType.DMA,
    ],
    compiler_params=pltpu.CompilerParams(needs_layout_passes=False),
)
def _sc_onehot(labels_hbm, out_hbm, buf_a, buf_b, lab_v, sem_a, sem_b):
    cid = lax.axis_index("c")
    sid = lax.axis_index("s")
    wid = sid * 2 + cid

    start = wid * _BASE + jnp.minimum(wid, _EXTRA)
    nch = jnp.where(wid < _EXTRA, _BASE + 1, _BASE)
    row0 = start * _CHUNK
    wbase = jnp.minimum(row0, _N - _LBUF)
    loff = row0 - wbase
    pltpu.sync_copy(labels_hbm.at[pl.ds(wbase, _LBUF)], lab_v)

    def _zero(i, carry):
        buf_a[pl.ds(i * 16, 16)] = jnp.zeros((16,), jnp.float32)
        buf_b[pl.ds(i * 16, 16)] = jnp.zeros((16,), jnp.float32)
        return carry

    lax.fori_loop(0, _CW // 16, _zero, 0)

    ones = jnp.ones((16,), jnp.float32)
    zeros = jnp.zeros((16,), jnp.float32)
    iota = lax.iota(jnp.int32, 16)

    def _scatter(buf, k, val):
        lbase = loff + k * _CHUNK
        for j in range(_CHUNK // 16):
            lab16 = lab_v[pl.ds(lbase + j * 16, 16)]
            lab16 = jnp.where(lab16 < 0, 55, lab16)
            idx = (j * 16 + iota) * _C + lab16
            plsc.store_scatter(buf, [idx], val)

    def _fire(buf, k, sem):
        pltpu.async_copy(buf, out_hbm.at[pl.ds((start + k) * _CW, _CW)], sem)

    def _drain(buf, k, sem):
        pltpu.make_async_copy(
            buf, out_hbm.at[pl.ds((start + k) * _CW, _CW)], sem
        ).wait()
        _scatter(buf, k, zeros)

    _scatter(buf_a, 0, ones)
    _fire(buf_a, 0, sem_a)

    def _pair(g, carry):
        k1 = 2 * g + 1
        k2 = 2 * g + 2

        @pl.when(k1 < nch)
        def _():
            _scatter(buf_b, k1, ones)
            _fire(buf_b, k1, sem_b)

        @pl.when(2 * g < nch)
        def _():
            _drain(buf_a, 2 * g, sem_a)

        @pl.when(k2 < nch)
        def _():
            _scatter(buf_a, k2, ones)
            _fire(buf_a, k2, sem_a)

        @pl.when(k1 < nch)
        def _():
            _drain(buf_b, k1, sem_b)

        return carry

    lax.fori_loop(0, _MAXC // 2, _pair, 0)


def kernel(x, labels):
    out = _sc_onehot(labels)
    return out.reshape(_N, _C)


# speedup vs baseline: 2.5133x; 2.4966x over previous
import functools

import jax
import jax.numpy as jnp
from jax import lax
from jax.experimental import pallas as pl
from jax.experimental.pallas import tpu as pltpu
from jax.experimental.pallas import tpu_sc as plsc

_N = 500000
_C = 100
_CHUNK = 400
_CW = _CHUNK * _C
_NCH = _N // _CHUNK
_NW = 32
_BASE = _NCH // _NW
_EXTRA = _NCH - _BASE * _NW
_MAXC = _BASE + 1
_LBUF = _MAXC * _CHUNK

_mesh = plsc.VectorSubcoreMesh(core_axis_name="c", subcore_axis_name="s")


@functools.partial(
    pl.kernel,
    mesh=_mesh,
    out_type=jax.ShapeDtypeStruct((_N, _C), jnp.float32),
    scratch_types=[
        pltpu.VMEM((_CHUNK, _C), jnp.float32),
        pltpu.VMEM((_CHUNK, _C), jnp.float32),
        pltpu.VMEM((_LBUF,), jnp.int32),
        pltpu.SemaphoreType.DMA,
        pltpu.SemaphoreType.DMA,
    ],
    compiler_params=pltpu.CompilerParams(needs_layout_passes=False),
)
def _sc_onehot(labels_hbm, out_hbm, buf_a, buf_b, lab_v, sem_a, sem_b):
    cid = lax.axis_index("c")
    sid = lax.axis_index("s")
    wid = sid * 2 + cid

    start = wid * _BASE + jnp.minimum(wid, _EXTRA)
    nch = jnp.where(wid < _EXTRA, _BASE + 1, _BASE)
    row0 = start * _CHUNK
    wbase = jnp.minimum(row0, _N - _LBUF)
    loff = row0 - wbase
    pltpu.sync_copy(labels_hbm.at[pl.ds(wbase, _LBUF)], lab_v)

    zero16 = jnp.zeros((16,), jnp.float32)

    def _zero(r, carry):
        for c in (0, 16, 32, 48, 64, 80, _C - 16):
            buf_a[r, pl.ds(c, 16)] = zero16
            buf_b[r, pl.ds(c, 16)] = zero16
        return carry

    lax.fori_loop(0, _CHUNK, _zero, 0)

    ones = jnp.ones((16,), jnp.float32)
    zeros = jnp.zeros((16,), jnp.float32)
    iota = lax.iota(jnp.int32, 16)

    def _scatter(buf, k, val):
        lbase = loff + k * _CHUNK
        for j in range(_CHUNK // 16):
            lab16 = lab_v[pl.ds(lbase + j * 16, 16)]
            lab16 = jnp.where(lab16 < 0, 55, lab16)
            row = j * 16 + iota
            plsc.store_scatter(buf, [row, lab16], val)

    def _fire(buf, k, sem):
        pltpu.async_copy(
            buf, out_hbm.at[pl.ds((start + k) * _CHUNK, _CHUNK)], sem
        )

    def _drain(buf, k, sem):
        pltpu.make_async_copy(
            buf, out_hbm.at[pl.ds((start + k) * _CHUNK, _CHUNK)], sem
        ).wait()
        _scatter(buf, k, zeros)

    _scatter(buf_a, 0, ones)
    _fire(buf_a, 0, sem_a)

    def _pair(g, carry):
        k1 = 2 * g + 1
        k2 = 2 * g + 2

        @pl.when(k1 < nch)
        def _():
            _scatter(buf_b, k1, ones)
            _fire(buf_b, k1, sem_b)

        @pl.when(2 * g < nch)
        def _():
            _drain(buf_a, 2 * g, sem_a)

        @pl.when(k2 < nch)
        def _():
            _scatter(buf_a, k2, ones)
            _fire(buf_a, k2, sem_a)

        @pl.when(k1 < nch)
        def _():
            _drain(buf_b, k1, sem_b)

        return carry

    lax.fori_loop(0, _MAXC // 2, _pair, 0)


def kernel(x, labels):
    return _sc_onehot(labels)


# speedup vs baseline: 8.3869x; 3.3370x over previous
import functools

import jax
import jax.numpy as jnp
from jax import lax
from jax.experimental import pallas as pl
from jax.experimental.pallas import tpu as pltpu
from jax.experimental.pallas import tpu_sc as plsc

_N = 500000
_C = 100
_COVER = (_N // 128) * 128
_W = 256
_FULL = _COVER // _W
_NW = 32
_CB = _FULL // _NW
_CE = _FULL - _CB * _NW
_NB = 4
_LBUF = 16384

_mesh = plsc.VectorSubcoreMesh(core_axis_name="c", subcore_axis_name="s")


@functools.partial(
    pl.kernel,
    mesh=_mesh,
    out_type=jax.ShapeDtypeStruct((_C, _N), jnp.float32),
    scratch_types=[
        pltpu.VMEM((_NB, _C, _W), jnp.float32),
        pltpu.VMEM((_LBUF,), jnp.int32),
        pltpu.SemaphoreType.DMA,
        pltpu.SemaphoreType.DMA,
        pltpu.SemaphoreType.DMA,
        pltpu.SemaphoreType.DMA,
        pltpu.SemaphoreType.DMA,
    ],
    compiler_params=pltpu.CompilerParams(needs_layout_passes=False),
)
def _sc_onehot_t(labels_hbm, out_hbm, bufs, lab_v, sem_l, s0, s1, s2, s3):
    sems = (s0, s1, s2, s3)
    cid = lax.axis_index("c")
    sid = lax.axis_index("s")
    wid = sid * 2 + cid

    cstart = wid * _CB + jnp.minimum(wid, _CE)
    nch = jnp.where(wid < _CE, _CB + 1, _CB)
    col0 = cstart * _W
    wbase = jnp.minimum(col0, _N - _LBUF)
    loff = col0 - wbase
    pltpu.async_copy(labels_hbm.at[pl.ds(wbase, _LBUF)], lab_v, sem_l)

    zero16 = jnp.zeros((16,), jnp.float32)

    def _zero(r, carry):
        for b in range(_NB):
            for g in range(_W // 16):
                bufs[b, r, pl.ds(g * 16, 16)] = zero16
        return carry

    lax.fori_loop(0, _C, _zero, 0)
    pltpu.make_async_copy(labels_hbm.at[pl.ds(wbase, _LBUF)], lab_v, sem_l).wait()

    ones = jnp.ones((16,), jnp.float32)
    zeros = jnp.zeros((16,), jnp.float32)
    iota = lax.iota(jnp.int32, 16)

    def _scatter(b, k, val):
        lbase = loff + k * _W
        for g in range(_W // 16):
            lab16 = lab_v[pl.ds(lbase + g * 16, 16)]
            lab16 = jnp.where(lab16 < 0, 55, lab16)
            col16 = g * 16 + iota
            plsc.store_scatter(bufs.at[b], [lab16, col16], val)

    _M = (_CB + 1 + _NB + (_NB - 1)) // _NB

    def _phase(m, carry):
        for j in range(_NB):
            s = _NB * m + j
            b = j

            @pl.when((s >= _NB) & (s - _NB < nch))
            def _():
                k = s - _NB
                pltpu.make_async_copy(
                    bufs.at[b],
                    out_hbm.at[:, pl.ds((cstart + k) * _W, _W)],
                    sems[b],
                ).wait()
                _scatter(b, k, zeros)

            @pl.when(s < nch)
            def _():
                _scatter(b, s, ones)
                pltpu.async_copy(
                    bufs.at[b],
                    out_hbm.at[:, pl.ds((cstart + s) * _W, _W)],
                    sems[b],
                )

        return carry

    lax.fori_loop(0, _M, _phase, 0)


def _tail_body(lab_ref, out_ref):
    lab = lab_ref[...]
    lab = jnp.where(lab < 0, 55, lab)
    iota_c = lax.broadcasted_iota(jnp.int32, (_C, _N - _COVER), 0)
    out_ref[...] = (iota_c == lab).astype(jnp.float32)


_tail_onehot = pl.pallas_call(
    _tail_body,
    out_shape=jax.ShapeDtypeStruct((_C, _N - _COVER), jnp.float32),
)


def kernel(x, labels):
    out_t = _sc_onehot_t(labels)
    tail = _tail_onehot(labels[_COVER:].reshape(1, _N - _COVER))
    out_t = lax.dynamic_update_slice(out_t, tail, (0, _COVER))
    return out_t.T


# speedup vs baseline: 8.3960x; 1.0011x over previous
import functools

import jax
import jax.numpy as jnp
from jax import lax
from jax.experimental import pallas as pl
from jax.experimental.pallas import tpu as pltpu
from jax.experimental.pallas import tpu_sc as plsc

_N = 500000
_C = 100
_COVER = (_N // 128) * 128
_W = 256
_FULL = _COVER // _W
_NW = 32
_CB = _FULL // _NW
_CE = _FULL - _CB * _NW
_NB = 4
_LBUF = 16384

_mesh = plsc.VectorSubcoreMesh(core_axis_name="c", subcore_axis_name="s")


@functools.partial(
    pl.kernel,
    mesh=_mesh,
    out_type=jax.ShapeDtypeStruct((_C, _N), jnp.float32),
    scratch_types=[
        pltpu.VMEM((_NB, _C, _W), jnp.float32),
        pltpu.VMEM((_LBUF,), jnp.int32),
        pltpu.SemaphoreType.DMA,
        pltpu.SemaphoreType.DMA,
        pltpu.SemaphoreType.DMA,
        pltpu.SemaphoreType.DMA,
        pltpu.SemaphoreType.DMA,
    ],
    compiler_params=pltpu.CompilerParams(
        needs_layout_passes=False, skip_device_barrier=True
    ),
)
def _sc_onehot_t(labels_hbm, out_hbm, bufs, lab_v, sem_l, s0, s1, s2, s3):
    sems = (s0, s1, s2, s3)
    cid = lax.axis_index("c")
    sid = lax.axis_index("s")
    wid = sid * 2 + cid

    cstart = wid * _CB + jnp.minimum(wid, _CE)
    nch = jnp.where(wid < _CE, _CB + 1, _CB)
    col0 = cstart * _W
    wbase = jnp.minimum(col0, _N - _LBUF)
    loff = col0 - wbase
    pltpu.async_copy(labels_hbm.at[pl.ds(wbase, _LBUF)], lab_v, sem_l)

    zero16 = jnp.zeros((16,), jnp.float32)

    def _zero(r, carry):
        for b in range(_NB):
            for g in range(_W // 16):
                bufs[b, r, pl.ds(g * 16, 16)] = zero16
        return carry

    lax.fori_loop(0, _C, _zero, 0)
    pltpu.make_async_copy(labels_hbm.at[pl.ds(wbase, _LBUF)], lab_v, sem_l).wait()

    ones = jnp.ones((16,), jnp.float32)
    zeros = jnp.zeros((16,), jnp.float32)
    iota = lax.iota(jnp.int32, 16)

    def _scatter(b, k, val):
        lbase = loff + k * _W
        for g in range(_W // 16):
            lab16 = lab_v[pl.ds(lbase + g * 16, 16)]
            lab16 = jnp.where(lab16 < 0, 55, lab16)
            col16 = g * 16 + iota
            plsc.store_scatter(bufs.at[b], [lab16, col16], val)

    _M = (_CB + 1 + _NB + (_NB - 1)) // _NB

    def _phase(m, carry):
        for j in range(_NB):
            s = _NB * m + j
            b = j

            @pl.when((s >= _NB) & (s - _NB < nch))
            def _():
                k = s - _NB
                pltpu.make_async_copy(
                    bufs.at[b],
                    out_hbm.at[:, pl.ds((cstart + k) * _W, _W)],
                    sems[b],
                ).wait()
                _scatter(b, k, zeros)

            @pl.when(s < nch)
            def _():
                _scatter(b, s, ones)
                pltpu.async_copy(
                    bufs.at[b],
                    out_hbm.at[:, pl.ds((cstart + s) * _W, _W)],
                    sems[b],
                )

        return carry

    lax.fori_loop(0, _M, _phase, 0)


def _tail_body(lab_ref, out_ref):
    lab = lab_ref[...]
    lab = jnp.where(lab < 0, 55, lab)
    iota_c = lax.broadcasted_iota(jnp.int32, (_C, _N - _COVER), 0)
    out_ref[...] = (iota_c == lab).astype(jnp.float32)


_tail_onehot = pl.pallas_call(
    _tail_body,
    out_shape=jax.ShapeDtypeStruct((_C, _N - _COVER), jnp.float32),
)


def kernel(x, labels):
    out_t = _sc_onehot_t(labels)
    tail = _tail_onehot(labels[_COVER:].reshape(1, _N - _COVER))
    out_t = lax.dynamic_update_slice(out_t, tail, (0, _COVER))
    return out_t.T


# speedup vs baseline: 8.4834x; 1.0104x over previous
import functools

import jax
import jax.numpy as jnp
from jax import lax
from jax.experimental import pallas as pl
from jax.experimental.pallas import tpu as pltpu
from jax.experimental.pallas import tpu_sc as plsc

_N = 500000
_C = 100
_COVER = (_N // 128) * 128
_W = 256
_FULL = _COVER // _W
_NW = 32
_CB = _FULL // _NW
_CE = _FULL - _CB * _NW
_NB = 4
_LBUF = 16384

_mesh = plsc.VectorSubcoreMesh(core_axis_name="c", subcore_axis_name="s")


@functools.partial(
    pl.kernel,
    mesh=_mesh,
    out_type=jax.ShapeDtypeStruct((_C, _N), jnp.float32),
    scratch_types=[
        pltpu.VMEM((_NB, _C, _W), jnp.float32),
        pltpu.VMEM((_LBUF,), jnp.int32),
        pltpu.SemaphoreType.DMA,
        pltpu.SemaphoreType.DMA,
        pltpu.SemaphoreType.DMA,
        pltpu.SemaphoreType.DMA,
        pltpu.SemaphoreType.DMA,
    ],
    compiler_params=pltpu.CompilerParams(
        needs_layout_passes=False, skip_device_barrier=True
    ),
)
def _sc_onehot_t(labels_hbm, out_hbm, bufs, lab_v, sem_l, s0, s1, s2, s3):
    sems = (s0, s1, s2, s3)
    cid = lax.axis_index("c")
    sid = lax.axis_index("s")
    wid = sid * 2 + cid

    cstart = wid * _CB + jnp.minimum(wid, _CE)
    nch = jnp.where(wid < _CE, _CB + 1, _CB)
    col0 = cstart * _W
    wbase = jnp.minimum(col0, _N - _LBUF)
    loff = col0 - wbase
    pltpu.async_copy(labels_hbm.at[pl.ds(wbase, _LBUF)], lab_v, sem_l)

    zero16 = jnp.zeros((16,), jnp.float32)

    def _zero_buf(b):
        def _zero(r, carry):
            for g in range(_W // 16):
                bufs[b, r, pl.ds(g * 16, 16)] = zero16
            return carry

        lax.fori_loop(0, _C, _zero, 0)

    ones = jnp.ones((16,), jnp.float32)
    zeros = jnp.zeros((16,), jnp.float32)
    iota = lax.iota(jnp.int32, 16)

    def _scatter(b, k, val):
        lbase = loff + k * _W
        for g in range(_W // 16):
            lab16 = lab_v[pl.ds(lbase + g * 16, 16)]
            lab16 = jnp.where(lab16 < 0, 55, lab16)
            col16 = g * 16 + iota
            plsc.store_scatter(bufs.at[b], [lab16, col16], val)

    def _fire(b, k):
        pltpu.async_copy(
            bufs.at[b], out_hbm.at[:, pl.ds((cstart + k) * _W, _W)], sems[b]
        )

    _zero_buf(0)
    pltpu.make_async_copy(labels_hbm.at[pl.ds(wbase, _LBUF)], lab_v, sem_l).wait()
    _scatter(0, 0, ones)
    _fire(0, 0)
    for b in range(1, _NB):
        _zero_buf(b)
        _scatter(b, b, ones)
        _fire(b, b)

    _M = (_CB + 2 + (_NB - 1)) // _NB

    def _phase(m, carry):
        for j in range(_NB):
            s = _NB * m + j + _NB
            b = j

            @pl.when(s - _NB < nch)
            def _():
                k = s - _NB
                pltpu.make_async_copy(
                    bufs.at[b],
                    out_hbm.at[:, pl.ds((cstart + k) * _W, _W)],
                    sems[b],
                ).wait()
                _scatter(b, k, zeros)

            @pl.when(s < nch)
            def _():
                _scatter(b, s, ones)
                _fire(b, s)

        return carry

    lax.fori_loop(0, _M, _phase, 0)


def _tail_body(lab_ref, out_ref):
    lab = lab_ref[...]
    lab = jnp.where(lab < 0, 55, lab)
    iota_c = lax.broadcasted_iota(jnp.int32, (_C, _N - _COVER), 0)
    out_ref[...] = (iota_c == lab).astype(jnp.float32)


_tail_onehot = pl.pallas_call(
    _tail_body,
    out_shape=jax.ShapeDtypeStruct((_C, _N - _COVER), jnp.float32),
)


def kernel(x, labels):
    out_t = _sc_onehot_t(labels)
    tail = _tail_onehot(labels[_COVER:].reshape(1, _N - _COVER))
    out_t = lax.dynamic_update_slice(out_t, tail, (0, _COVER))
    return out_t.T
